# Initial kernel scaffold; baseline (speedup 1.0000x reference)
import jax
import jax.numpy as jnp
from jax.experimental import pallas as pl


def kernel(pair_dist, pair_i, pair_j, atom_batch):
    raise NotImplementedError("write your pallas kernel here")


# speedup vs baseline: 35.3102x; 35.3102x over previous
import functools

import jax
import jax.numpy as jnp
from jax import lax
from jax.experimental import pallas as pl
from jax.experimental.pallas import tpu as pltpu
from jax.experimental.pallas import tpu_sc as plsc

N_ATOMS = 10000
N_PAIRS = 640000
NC = 2
NS = 16
NW = NC * NS
PER_TILE = N_PAIRS // NW
CHUNK = 2000
NCHUNK = PER_TILE // CHUNK
LANES = 16


def _sc_pair_force(xs, ys, zs, pi, pj, out,
                   xb, yb, zb, ib, jb,
                   accx, accy, accz, afx, afy, afz, ihead):
    c = lax.axis_index("c")
    s = lax.axis_index("s")
    wid = s * NC + c
    base0 = wid * PER_TILE
    zero16 = jnp.zeros((LANES,), jnp.float32)

    def zero_body(k, carry):
        sl = pl.ds(k * LANES, LANES)
        accx[sl] = zero16
        accy[sl] = zero16
        accz[sl] = zero16
        afx[sl] = zero16
        afy[sl] = zero16
        afz[sl] = zero16
        return carry

    lax.fori_loop(0, N_ATOMS // LANES, zero_body, 0)

    pltpu.sync_copy(pi.at[pl.ds(0, N_ATOMS)], ihead)

    for ch in range(NCHUNK):
        b = base0 + ch * CHUNK
        pltpu.sync_copy(xs.at[pl.ds(b, CHUNK)], xb)
        pltpu.sync_copy(ys.at[pl.ds(b, CHUNK)], yb)
        pltpu.sync_copy(zs.at[pl.ds(b, CHUNK)], zb)
        pltpu.sync_copy(pi.at[pl.ds(b, CHUNK)], ib)
        pltpu.sync_copy(pj.at[pl.ds(b, CHUNK)], jb)

        def force_body(v, carry):
            sl = pl.ds(v * LANES, LANES)
            x = xb[sl]
            y = yb[sl]
            z = zb[sl]
            r2 = x * x + y * y + z * z + 0.01
            inv = 1.0 / r2
            inv3 = inv * inv * inv
            coef = inv3 * inv * (24.0 - 48.0 * inv3)
            fx = coef * x
            fy = coef * y
            fz = coef * z
            ii = ib[sl]
            jj = jb[sl]
            plsc.addupdate_scatter(accx, [ii], fx)
            plsc.addupdate_scatter(accy, [ii], fy)
            plsc.addupdate_scatter(accz, [ii], fz)
            plsc.addupdate_scatter(accx, [jj], -fx)
            plsc.addupdate_scatter(accy, [jj], -fy)
            plsc.addupdate_scatter(accz, [jj], -fz)
            return carry

        lax.fori_loop(0, CHUNK // LANES, force_body, 0)

    def stage2_body(k, carry):
        sl = pl.ds(k * LANES, LANES)
        idx = ihead[sl]
        plsc.addupdate_scatter(afx, [idx], accx[sl])
        plsc.addupdate_scatter(afy, [idx], accy[sl])
        plsc.addupdate_scatter(afz, [idx], accz[sl])
        return carry

    lax.fori_loop(0, N_ATOMS // LANES, stage2_body, 0)

    obase = wid * 3 * N_ATOMS
    pltpu.sync_copy(afx, out.at[pl.ds(obase, N_ATOMS)])
    pltpu.sync_copy(afy, out.at[pl.ds(obase + N_ATOMS, N_ATOMS)])
    pltpu.sync_copy(afz, out.at[pl.ds(obase + 2 * N_ATOMS, N_ATOMS)])


def _combine_body(x_ref, o_ref):
    o_ref[...] = jnp.sum(x_ref[...], axis=0)


@jax.jit
def kernel(pair_dist, pair_i, pair_j, atom_batch):
    xs = pair_dist[:, 0]
    ys = pair_dist[:, 1]
    zs = pair_dist[:, 2]

    mesh = plsc.VectorSubcoreMesh(core_axis_name="c", subcore_axis_name="s")
    sc_fn = pl.kernel(
        _sc_pair_force,
        out_type=jax.ShapeDtypeStruct((NW * 3 * N_ATOMS,), jnp.float32),
        mesh=mesh,
        compiler_params=pltpu.CompilerParams(needs_layout_passes=False),
        scratch_types=[
            pltpu.VMEM((CHUNK,), jnp.float32),
            pltpu.VMEM((CHUNK,), jnp.float32),
            pltpu.VMEM((CHUNK,), jnp.float32),
            pltpu.VMEM((CHUNK,), jnp.int32),
            pltpu.VMEM((CHUNK,), jnp.int32),
            pltpu.VMEM((N_ATOMS,), jnp.float32),
            pltpu.VMEM((N_ATOMS,), jnp.float32),
            pltpu.VMEM((N_ATOMS,), jnp.float32),
            pltpu.VMEM((N_ATOMS,), jnp.float32),
            pltpu.VMEM((N_ATOMS,), jnp.float32),
            pltpu.VMEM((N_ATOMS,), jnp.float32),
            pltpu.VMEM((N_ATOMS,), jnp.int32),
        ],
    )
    partials = sc_fn(xs, ys, zs, pair_i, pair_j).reshape(NW, 3, N_ATOMS)

    combined = pl.pallas_call(
        _combine_body,
        out_shape=jax.ShapeDtypeStruct((3, N_ATOMS), jnp.float32),
    )(partials)

    return combined.T
